# Initial kernel scaffold; baseline (speedup 1.0000x reference)
import jax
import jax.numpy as jnp
from jax.experimental import pallas as pl


def kernel(node_feat, edge_feat, edge_index, etype_ids, token_ids, W_node, b_node, W_edge, b_edge, graph_tok, sep_tok, order_tab, etype_tab, token_tab):
    raise NotImplementedError("write your pallas kernel here")


# speedup vs baseline: 1.6351x; 1.6351x over previous
import jax
import jax.numpy as jnp
from jax import lax
from jax.experimental import pallas as pl
from jax.experimental.pallas import tpu as pltpu
from jax.experimental.pallas import tpu_sc as plsc

B, N, E, D = 64, 512, 2048, 128
TOUT = N + E + 3
NC, NS, L = 2, 16, 16
NW = NC * NS
BPW = B // NW
C = 128
NODE_CHUNKS = N // C
EDGE_CHUNKS = E // C
ND = D // L


def _splat(x):
    return jnp.full((L,), x, jnp.int32)


def _body(node_feat, edge_feat, edge_index, etype_ids, token_ids,
          wn_hbm, we_hbm, nb_hbm, comb_hbm, spec_hbm, token_tab,
          out,
          wn_v, we_v, nb_v, comb_v, spec_v,
          featv, tidv, etv, srcv, dstv, cidxv,
          gbuf, tokv, nodebuf, tailv, sem):
    cid = lax.axis_index("c")
    sid = lax.axis_index("s")
    wid = sid * NC + cid

    pltpu.sync_copy(wn_hbm, wn_v)
    pltpu.sync_copy(we_hbm, we_v)
    pltpu.sync_copy(nb_hbm, nb_v)
    pltpu.sync_copy(comb_hbm, comb_v)
    pltpu.sync_copy(spec_hbm, spec_v)

    iota = lax.iota(jnp.int32, L)

    def batch_body(bi, _):
        b = wid * BPW + bi

        for d in range(ND):
            nodebuf[0, pl.ds(d * L, L)] = spec_v[0, pl.ds(d * L, L)]

        def node_chunk(ci, _):
            base = b * N + ci * C
            pltpu.sync_copy(node_feat.at[pl.ds(base, C)], featv)

            def node_row(r, _):
                f = [plsc.load_gather(featv, [_splat(r), _splat(k)])
                     for k in range(4)]
                for d in range(ND):
                    acc = nb_v[pl.ds(d * L, L)]
                    for k in range(4):
                        acc = acc + f[k] * wn_v[k, pl.ds(d * L, L)]
                    nodebuf[r + 1, pl.ds(d * L, L)] = acc
                return 0

            lax.fori_loop(0, C, node_row, 0)
            pltpu.sync_copy(nodebuf.at[pl.ds(0, C)],
                            out.at[b, pl.ds(ci * C, C)])
            for d in range(ND):
                nodebuf[0, pl.ds(d * L, L)] = nodebuf[C, pl.ds(d * L, L)]
            return 0

        lax.fori_loop(0, NODE_CHUNKS, node_chunk, 0)

        for d in range(ND):
            tokv[0, pl.ds(d * L, L)] = nodebuf[0, pl.ds(d * L, L)]

        def edge_chunk(ci, _):
            base = b * E + ci * C
            pltpu.sync_copy(token_ids.at[pl.ds(base, C)], tidv)
            pltpu.sync_copy(etype_ids.at[pl.ds(base, C)], etv)
            pltpu.sync_copy(edge_index.at[0, pl.ds(base, C)], srcv)
            pltpu.sync_copy(edge_index.at[1, pl.ds(base, C)], dstv)
            pltpu.sync_copy(edge_feat.at[pl.ds(base, C)], featv)
            pltpu.async_copy(token_tab.at[tidv], gbuf, sem).wait()

            def cidx_group(g, _):
                et16 = etv[pl.ds(g * L, L)]
                s16 = srcv[pl.ds(g * L, L)]
                d16 = dstv[pl.ds(g * L, L)]
                cidxv[pl.ds(g * L, L)] = (
                    et16 * 2 + (s16 == d16).astype(jnp.int32))
                return 0

            lax.fori_loop(0, C // L, cidx_group, 0)

            def edge_row(r, _):
                f = [plsc.load_gather(featv, [_splat(r), _splat(k)])
                     for k in range(4)]
                cs = plsc.load_gather(cidxv, [_splat(r)])
                for d in range(ND):
                    tab = plsc.load_gather(comb_v, [cs, iota + d * L])
                    acc = gbuf[r, pl.ds(d * L, L)] + tab
                    for k in range(4):
                        acc = acc + f[k] * we_v[k, pl.ds(d * L, L)]
                    tokv[r + 1, pl.ds(d * L, L)] = acc
                return 0

            lax.fori_loop(0, C, edge_row, 0)
            pltpu.sync_copy(tokv.at[pl.ds(0, C)],
                            out.at[b, pl.ds(N + ci * C, C)])
            for d in range(ND):
                tokv[0, pl.ds(d * L, L)] = tokv[C, pl.ds(d * L, L)]
            return 0

        lax.fori_loop(0, EDGE_CHUNKS, edge_chunk, 0)

        for d in range(ND):
            tailv[0, pl.ds(d * L, L)] = tokv[0, pl.ds(d * L, L)]
            tailv[1, pl.ds(d * L, L)] = spec_v[1, pl.ds(d * L, L)]
            tailv[2, pl.ds(d * L, L)] = spec_v[2, pl.ds(d * L, L)]
        pltpu.sync_copy(tailv, out.at[b, pl.ds(N + E, 3)])
        return 0

    lax.fori_loop(0, BPW, batch_body, 0)


def kernel(node_feat, edge_feat, edge_index, etype_ids, token_ids,
           W_node, b_node, W_edge, b_edge, graph_tok, sep_tok,
           order_tab, etype_tab, token_tab):
    edge_index = edge_index.astype(jnp.int32)
    etype_ids = etype_ids.astype(jnp.int32)
    token_ids = token_ids.astype(jnp.int32)

    wn = jnp.asarray(W_node.T, jnp.float32)
    we = jnp.asarray(W_edge.T, jnp.float32)
    nb = (b_node + order_tab[2]).astype(jnp.float32)
    comb = (etype_tab[:, None, :]
            + (order_tab[1:3] + b_edge)[None, :, :]).reshape(8, D)
    spec = jnp.concatenate([graph_tok, sep_tok, order_tab[0:1]], axis=0)

    mesh = plsc.VectorSubcoreMesh(core_axis_name="c", subcore_axis_name="s",
                                  num_cores=NC, num_subcores=NS)
    run = pl.kernel(
        _body,
        out_type=jax.ShapeDtypeStruct((B, TOUT, D), jnp.float32),
        mesh=mesh,
        compiler_params=pltpu.CompilerParams(needs_layout_passes=False),
        scratch_types=[
            pltpu.VMEM((4, D), jnp.float32),
            pltpu.VMEM((4, D), jnp.float32),
            pltpu.VMEM((D,), jnp.float32),
            pltpu.VMEM((8, D), jnp.float32),
            pltpu.VMEM((3, D), jnp.float32),
            pltpu.VMEM((C, 4), jnp.float32),
            pltpu.VMEM((C,), jnp.int32),
            pltpu.VMEM((C,), jnp.int32),
            pltpu.VMEM((C,), jnp.int32),
            pltpu.VMEM((C,), jnp.int32),
            pltpu.VMEM((C,), jnp.int32),
            pltpu.VMEM((C, D), jnp.float32),
            pltpu.VMEM((C + 1, D), jnp.float32),
            pltpu.VMEM((C + 1, D), jnp.float32),
            pltpu.VMEM((3, D), jnp.float32),
            pltpu.SemaphoreType.DMA,
        ],
    )
    return run(node_feat, edge_feat, edge_index, etype_ids, token_ids,
               wn, we, nb, comb, spec, token_tab)
